# Initial kernel scaffold; baseline (speedup 1.0000x reference)
import jax
import jax.numpy as jnp
from jax.experimental import pallas as pl


def kernel(x, W_bb, b_bb, W_rpn, b_rpn, W_cls, b_cls, W_reg, b_reg, W_fc, b_fc, W_hc, b_hc, W_hr, b_hr):
    raise NotImplementedError("write your pallas kernel here")


# speedup vs baseline: 4.4892x; 4.4892x over previous
import functools

import jax
import jax.numpy as jnp
import numpy as np
from jax import lax
from jax.experimental import pallas as pl
from jax.experimental.pallas import tpu as pltpu

try:
    from jax.experimental.pallas import tpu_sc as plsc
    _HAS_SC = True
except ImportError:
    plsc = None
    _HAS_SC = False

IMG = 512
STRIDE = 16
FS = 32
A = 9
NPOS = FS * FS
NANCH = NPOS * A
PRE_NMS = 1000
POST_NMS = 300
NMS_THR = 0.7
POOL = 7
NUM_CLASSES = 21
CF = 256
NTOP = 1024
NPOST = 304
ROI_B = 16384


def _anchor_comps():
    scales = [64.0, 128.0, 256.0]
    ratios = [0.5, 1.0, 2.0]
    base = []
    for s in scales:
        for r in ratios:
            w = s * np.sqrt(r)
            h = s / np.sqrt(r)
            base.append([-w / 2.0, -h / 2.0, w / 2.0, h / 2.0])
    base = np.asarray(base, np.float32)
    c = (np.arange(FS, dtype=np.float32) + 0.5) * STRIDE
    cx, cy = np.meshgrid(c, c)
    shifts = np.stack([cx, cy, cx, cy], axis=-1).reshape(-1, 1, 4)
    anch = (shifts + base[None]).reshape(-1, 4)
    wa = (anch[:, 2] - anch[:, 0]).reshape(NPOS, A)
    ha = (anch[:, 3] - anch[:, 1]).reshape(NPOS, A)
    xa = (anch[:, 0].reshape(NPOS, A) + 0.5 * wa)
    ya = (anch[:, 1].reshape(NPOS, A) + 0.5 * ha)
    return (jnp.asarray(wa), jnp.asarray(ha), jnp.asarray(xa), jnp.asarray(ya))


def _kb1_body(s_ref, ts_ref, ti_ref):
    s0 = s_ref[:]
    flat_iota = (lax.broadcasted_iota(jnp.int32, (8, 1152), 0) * 1152
                 + lax.broadcasted_iota(jnp.int32, (8, 1152), 1))
    rank_iota = (lax.broadcasted_iota(jnp.int32, (8, 128), 0) * 128
                 + lax.broadcasted_iota(jnp.int32, (8, 128), 1))

    def body(t, carry):
        s, ts, ti = carry
        m = jnp.max(s)
        qi = jnp.min(jnp.where(s == m, flat_iota, jnp.int32(1 << 30)))
        sel = rank_iota == t
        ts = jnp.where(sel, m, ts)
        ti = jnp.where(sel, qi, ti)
        s = jnp.where(flat_iota == qi, -2e9, s)
        return s, ts, ti

    _, ts, ti = lax.fori_loop(
        0, PRE_NMS, body,
        (s0, jnp.full((8, 128), -2e9, jnp.float32),
         jnp.zeros((8, 128), jnp.int32)))
    ts_ref[:] = ts
    ti_ref[:] = ti


def _run_b1(s, interpret=False):
    return pl.pallas_call(
        _kb1_body,
        out_shape=(jax.ShapeDtypeStruct((8, 128), jnp.float32),
                   jax.ShapeDtypeStruct((8, 128), jnp.int32)),
        interpret=interpret,
    )(s)


def _kb2_body(ti_ref, x1_ref, y1_ref, x2_ref, y2_ref,
              ox1_ref, oy1_ref, ox2_ref, oy2_ref):
    ti = ti_ref[:]
    r = ti // 1152
    l = ti - r * 1152
    lane_mask = (lax.broadcasted_iota(jnp.int32, (NTOP, 1152), 1)
                 == l).astype(jnp.float32)
    for src, dst in ((x1_ref, ox1_ref), (y1_ref, oy1_ref),
                     (x2_ref, ox2_ref), (y2_ref, oy2_ref)):
        s = src[:]
        g = jnp.zeros((NTOP, 1152), jnp.float32)
        for rr in range(8):
            g = jnp.where(r == rr, s[rr:rr + 1, :], g)
        dst[:] = jnp.sum(g * lane_mask, axis=1, keepdims=True)


def _run_b2(ti_col, x1, y1, x2, y2, interpret=False):
    shc = jax.ShapeDtypeStruct((NTOP, 1), jnp.float32)
    return pl.pallas_call(
        _kb2_body,
        out_shape=(shc, shc, shc, shc),
        interpret=interpret,
    )(ti_col, x1, y1, x2, y2)


def _kb3_body(x1c_ref, y1c_ref, x2c_ref, y2c_ref,
              x1r_ref, y1r_ref, x2r_ref, y2r_ref, tsr_ref,
              pi_ref, m_scr):
    x1c, y1c, x2c, y2c = x1c_ref[:], y1c_ref[:], x2c_ref[:], y2c_ref[:]
    x1r, y1r, x2r, y2r = x1r_ref[:], y1r_ref[:], x2r_ref[:], y2r_ref[:]
    area_c = (x2c - x1c) * (y2c - y1c)
    area_r = (x2r - x1r) * (y2r - y1r)
    ltx = jnp.maximum(x1c, x1r)
    lty = jnp.maximum(y1c, y1r)
    rbx = jnp.minimum(x2c, x2r)
    rby = jnp.minimum(y2c, y2r)
    wx = jnp.maximum(rbx - ltx, 0.0)
    wy = jnp.maximum(rby - lty, 0.0)
    inter = wx * wy
    iou = inter / (area_c + area_r - inter + 1e-6)
    row_i = lax.broadcasted_iota(jnp.int32, (NTOP, NTOP), 0)
    col_i = lax.broadcasted_iota(jnp.int32, (NTOP, NTOP), 1)
    m_scr[:] = ((iou > NMS_THR) & (col_i > row_i)).astype(jnp.float32)

    lane_i = lax.broadcasted_iota(jnp.int32, (1, NTOP), 1)
    keep0 = (lane_i < PRE_NMS).astype(jnp.float32)

    def outer(g, keep):
        block = m_scr[pl.ds(g * 8, 8), :]
        for k in range(8):
            i = g * 8 + k
            row = block[k:k + 1, :]
            keep_i = jnp.sum(jnp.where(lane_i == i, keep, 0.0))
            supp = row * keep * keep_i
            keep = keep * (1.0 - supp)
        return keep

    keep = lax.fori_loop(0, NTOP // 8, outer, keep0)

    ks = jnp.where((keep > 0.0) & (lane_i < PRE_NMS), tsr_ref[:], -1e9)
    rank_iota = (lax.broadcasted_iota(jnp.int32, (8, 128), 0) * 128
                 + lax.broadcasted_iota(jnp.int32, (8, 128), 1))

    def post(t, carry):
        ks_c, pi = carry
        m = jnp.max(ks_c)
        qi = jnp.min(jnp.where(ks_c == m, lane_i, jnp.int32(1 << 30)))
        pi = jnp.where(rank_iota == t, qi, pi)
        ks_c = jnp.where(lane_i == qi, -2e9, ks_c)
        return ks_c, pi

    _, pi = lax.fori_loop(0, POST_NMS, post,
                          (ks, jnp.zeros((8, 128), jnp.int32)))
    pi_ref[:] = pi


def _run_b3(x1c, y1c, x2c, y2c, x1r, y1r, x2r, y2r, tsr, interpret=False):
    return pl.pallas_call(
        _kb3_body,
        out_shape=jax.ShapeDtypeStruct((8, 128), jnp.int32),
        scratch_shapes=[pltpu.VMEM((NTOP, NTOP), jnp.float32)],
        interpret=interpret,
    )(x1c, y1c, x2c, y2c, x1r, y1r, x2r, y2r, tsr)


def _kb4_body(pi_ref, x1r_ref, y1r_ref, x2r_ref, y2r_ref, roi_ref):
    pi = pi_ref[:]
    onehot = (lax.broadcasted_iota(jnp.int32, (NPOST, NTOP), 1)
              == pi).astype(jnp.float32)
    px1 = jnp.sum(onehot * x1r_ref[:], axis=1, keepdims=True)
    py1 = jnp.sum(onehot * y1r_ref[:], axis=1, keepdims=True)
    px2 = jnp.sum(onehot * x2r_ref[:], axis=1, keepdims=True)
    py2 = jnp.sum(onehot * y2r_ref[:], axis=1, keepdims=True)
    q = lax.broadcasted_iota(jnp.int32, (1, POOL * POOL), 1)
    ci = (q // POOL).astype(jnp.float32)
    cj = (q - (q // POOL) * POOL).astype(jnp.float32)
    gy = py1 + (ci + 0.5) * (py2 - py1) / float(POOL)
    gx = px1 + (cj + 0.5) * (px2 - px1) / float(POOL)
    iy = jnp.clip((gy / float(STRIDE)).astype(jnp.int32), 0, FS - 1)
    ix = jnp.clip((gx / float(STRIDE)).astype(jnp.int32), 0, FS - 1)
    roi_ref[:] = iy * FS + ix


def _run_b4(pi_col, x1r, y1r, x2r, y2r, interpret=False):
    return pl.pallas_call(
        _kb4_body,
        out_shape=jax.ShapeDtypeStruct((NPOST, POOL * POOL), jnp.int32),
        interpret=interpret,
    )(pi_col, x1r, y1r, x2r, y2r)


def _make_sc_gather():
    info = plsc.get_sparse_core_info()
    nc, ns = info.num_cores, info.num_subcores
    nw = nc * ns
    b_per_w = ROI_B // nw
    ch = 128
    nch = b_per_w // ch
    mesh = plsc.VectorSubcoreMesh(core_axis_name="c", subcore_axis_name="s")

    @functools.partial(
        pl.kernel, mesh=mesh,
        out_type=jax.ShapeDtypeStruct((ROI_B, CF), jnp.float32),
        scratch_types=[
            pltpu.VMEM((ch,), jnp.int32),
            pltpu.VMEM((ch, CF), jnp.float32),
            pltpu.SemaphoreType.DMA,
        ],
    )
    def roi_gather(table_hbm, idx_hbm, out_hbm, idx_v, rows_v, sem):
        wid = lax.axis_index("s") * nc + lax.axis_index("c")
        for t in range(nch):
            base = wid * b_per_w + t * ch
            pltpu.sync_copy(idx_hbm.at[pl.ds(base, ch)], idx_v)
            pltpu.async_copy(table_hbm.at[idx_v], rows_v, sem).wait()
            pltpu.sync_copy(rows_v, out_hbm.at[pl.ds(base, ch)])

    return roi_gather


def _kd_body(p_ref, w_ref, b_ref, o_ref):
    k = pl.program_id(0)

    @pl.when(k == 0)
    def _init():
        o_ref[:] = jnp.zeros_like(o_ref)

    o_ref[:] += jnp.dot(p_ref[:], w_ref[:], preferred_element_type=jnp.float32)

    @pl.when(k == pl.num_programs(0) - 1)
    def _fin():
        o_ref[:] = jnp.maximum(o_ref[:] + b_ref[:], 0.0)


def _run_d(pooled, wfc, bfc, interpret=False):
    ksplit = 7
    kblk = pooled.shape[1] // ksplit
    return pl.pallas_call(
        _kd_body,
        grid=(ksplit,),
        in_specs=[
            pl.BlockSpec((NPOST, kblk), lambda k: (0, k)),
            pl.BlockSpec((kblk, 1024), lambda k: (k, 0)),
            pl.BlockSpec((1, 1024), lambda k: (0, 0)),
        ],
        out_specs=pl.BlockSpec((NPOST, 1024), lambda k: (0, 0)),
        out_shape=jax.ShapeDtypeStruct((NPOST, 1024), jnp.float32),
        interpret=interpret,
    )(pooled, wfc, bfc)


def _ke_body(h_ref, wc_ref, bc_ref, wr_ref, br_ref, c_ref, r_ref):
    h = h_ref[:]
    c_ref[:] = jnp.dot(h, wc_ref[:], preferred_element_type=jnp.float32) + bc_ref[:]
    r_ref[:] = jnp.dot(h, wr_ref[:], preferred_element_type=jnp.float32) + br_ref[:]


def _run_e(h2, whc, bhc, whr, bhr, interpret=False):
    return pl.pallas_call(
        _ke_body,
        out_shape=(jax.ShapeDtypeStruct((NPOST, NUM_CLASSES), jnp.float32),
                   jax.ShapeDtypeStruct((NPOST, 4 * NUM_CLASSES), jnp.float32)),
        interpret=interpret,
    )(h2, whc, bhc, whr, bhr)


def _pipeline(x, W_bb, b_bb, W_rpn, b_rpn, W_cls, b_cls, W_reg, b_reg,
              W_fc, b_fc, W_hc, b_hc, W_hr, b_hr,
              interpret=False, roi_gather_fn=None):
    def _conv2d(v, w, b, stride, padding):
        y = lax.conv_general_dilated(v, w, (stride, stride), padding,
                                     dimension_numbers=('NCHW', 'OIHW', 'NCHW'))
        return y + b[None, :, None, None]

    wa, ha, xa, ya = _anchor_comps()
    feat4 = jax.nn.relu(_conv2d(x, W_bb, b_bb, STRIDE, 'VALID'))
    hmap = jax.nn.relu(_conv2d(feat4, W_rpn, b_rpn, 1, 'SAME'))
    cls_rpn = _conv2d(hmap, W_cls, b_cls, 1, 'SAME')
    reg_rpn = _conv2d(hmap, W_reg, b_reg, 1, 'SAME')
    scores = jax.nn.sigmoid(jnp.transpose(cls_rpn[0], (1, 2, 0)).reshape(NPOS, A))
    deltas = jnp.transpose(reg_rpn[0].reshape(A, 4, FS, FS), (2, 3, 0, 1))
    dx, dy = deltas[..., 0], deltas[..., 1]
    dw = jnp.clip(deltas[..., 2], -4.0, 4.0)
    dh = jnp.clip(deltas[..., 3], -4.0, 4.0)
    dx = dx.reshape(NPOS, A)
    dy = dy.reshape(NPOS, A)
    dw = dw.reshape(NPOS, A)
    dh = dh.reshape(NPOS, A)
    xc = xa + dx * wa
    yc = ya + dy * ha
    w_ = wa * jnp.exp(dw)
    h_ = ha * jnp.exp(dh)
    bx1 = jnp.clip(xc - 0.5 * w_, 0.0, float(IMG))
    by1 = jnp.clip(yc - 0.5 * h_, 0.0, float(IMG))
    bx2 = jnp.clip(xc + 0.5 * w_, 0.0, float(IMG))
    by2 = jnp.clip(yc + 0.5 * h_, 0.0, float(IMG))
    feat = feat4[0].transpose(1, 2, 0).reshape(NPOS, CF)

    s8 = scores.reshape(8, 1152)
    ts, ti = _run_b1(s8, interpret=interpret)

    ti_col = ti.reshape(NTOP, 1)
    tx1, ty1, tx2, ty2 = _run_b2(
        ti_col, bx1.reshape(8, 1152), by1.reshape(8, 1152),
        bx2.reshape(8, 1152), by2.reshape(8, 1152), interpret=interpret)

    x1r, y1r = tx1.reshape(1, NTOP), ty1.reshape(1, NTOP)
    x2r, y2r = tx2.reshape(1, NTOP), ty2.reshape(1, NTOP)
    pi = _run_b3(tx1, ty1, tx2, ty2, x1r, y1r, x2r, y2r,
                 ts.reshape(1, NTOP), interpret=interpret)

    pi_col = pi.reshape(NTOP, 1)[:NPOST]
    roi = _run_b4(pi_col, x1r, y1r, x2r, y2r, interpret=interpret)

    roi_flat = roi[:POST_NMS].reshape(POST_NMS * POOL * POOL)
    roi_idx = jnp.concatenate(
        [roi_flat, jnp.zeros((ROI_B - POST_NMS * POOL * POOL,), jnp.int32)])

    if roi_gather_fn is None:
        roi_gather_fn = _make_sc_gather()
    rows = roi_gather_fn(feat, roi_idx)

    pooled = rows[:POST_NMS * POOL * POOL].reshape(POST_NMS, POOL * POOL * CF)
    pooled = jnp.concatenate(
        [pooled, jnp.zeros((NPOST - POST_NMS, POOL * POOL * CF), jnp.float32)])

    wfc = W_fc.reshape(CF, POOL, POOL, 1024).transpose(1, 2, 0, 3)
    wfc = wfc.reshape(CF * POOL * POOL, 1024)
    h2 = _run_d(pooled, wfc, b_fc.reshape(1, 1024), interpret=interpret)

    cls, reg = _run_e(h2, W_hc, b_hc.reshape(1, NUM_CLASSES),
                      W_hr, b_hr.reshape(1, 4 * NUM_CLASSES),
                      interpret=interpret)
    return cls[:POST_NMS], reg[:POST_NMS]


def kernel(x, W_bb, b_bb, W_rpn, b_rpn, W_cls, b_cls, W_reg, b_reg,
           W_fc, b_fc, W_hc, b_hc, W_hr, b_hr):
    return _pipeline(x, W_bb, b_bb, W_rpn, b_rpn, W_cls, b_cls, W_reg, b_reg,
                     W_fc, b_fc, W_hc, b_hc, W_hr, b_hr)
